# baseline (speedup 1.0000x reference)
import functools

import jax
import jax.numpy as jnp
from jax import lax
from jax.experimental import pallas as pl
from jax.experimental.pallas import tpu as pltpu
from jax.experimental.pallas import tpu_sc as plsc

V = 100
VP = 128
D = 128
B = 16384
VARIATIONAL_WEIGHT = 0.001

NC = 2
NS = 16
L = 16
NW = NC * NS
B_PER_W = B // NW
CHUNK = 128
N_CHUNKS = B_PER_W // CHUNK
R_PER_S = 8
NS_FULL = V // R_PER_S


def _sc_body(idx_hbm, eps_hbm, mean_hbm, lv_hbm, code_hbm, kld_hbm,
             sm_sh, g_sh,
             sm_t, g_v, idx_v, kld_v,
             eps_rv, code_rv,
             a_st, b_st, p_st, g_st,
             sem_i0, sem_i1, sem_o0, sem_o1, sem_x):
    sid = lax.axis_index("s")
    wid = sid * NC + lax.axis_index("c")
    lane = lax.iota(jnp.int32, L)
    wbase = wid * B_PER_W
    CD = CHUNK * D

    cp_idx = pltpu.async_copy(idx_hbm.at[pl.ds(wbase, B_PER_W)], idx_v, sem_x)
    cp_eps0 = pltpu.async_copy(eps_hbm.at[pl.ds(wbase * D, CD)],
                               eps_rv.at[pl.ds(0, CD)], sem_i0)

    def prelude(trow, nrows):
        cp_a = pltpu.async_copy(lv_hbm.at[pl.ds(trow * D, nrows * D)],
                                a_st.at[pl.ds(0, nrows * D)], sem_x)
        cp_b = pltpu.async_copy(mean_hbm.at[pl.ds(trow * D, nrows * D)],
                                b_st.at[pl.ds(0, nrows * D)], sem_x)
        cp_a.wait()
        cp_b.wait()

        def prow(r2, g_acc):
            acc = jnp.zeros((L,), jnp.float32)
            for j in range(D // L):
                sl = pl.ds(r2 * D + j * L, L)
                lv = a_st[sl]
                m = b_st[sl]
                s = jnp.exp(0.5 * lv)
                p_st[pl.ds(r2 * D + j * L, L)] = plsc.bitcast(
                    plsc.pack(s, m, format=plsc.PackFormat.INTERLEAVED),
                    jnp.int32)
                acc = acc + (1.0 + lv - m * m - s * s)
            for _ in range(4):
                acc = acc + lax.rev(acc, (0,))
            return jnp.where(lane == r2, acc * (-0.5 * VARIATIONAL_WEIGHT),
                             g_acc)

        g_st[...] = lax.fori_loop(0, nrows, prow,
                                  jnp.zeros((L,), jnp.float32))
        pltpu.sync_copy(p_st.at[pl.ds(0, nrows * D)],
                        sm_sh.at[pl.ds(trow * D, nrows * D)])
        pltpu.sync_copy(g_st.at[pl.ds(0, R_PER_S)],
                        g_sh.at[pl.ds(trow, R_PER_S)])

    @pl.when(sid < NS_FULL)
    def _():
        prelude(sid * R_PER_S, R_PER_S)

    @pl.when(sid == NS_FULL)
    def _():
        prelude(NS_FULL * R_PER_S, V - NS_FULL * R_PER_S)

    plsc.subcore_barrier()
    cp_sm = pltpu.async_copy(sm_sh, sm_t, sem_x)
    cp_g = pltpu.async_copy(g_sh, g_v, sem_x)
    cp_idx.wait()
    cp_sm.wait()
    cp_g.wait()

    def drain_in(sem):
        pltpu.make_async_copy(eps_hbm.at[pl.ds(0, CD)],
                              eps_rv.at[pl.ds(0, CD)], sem).wait()

    def drain_out(sem):
        pltpu.make_async_copy(code_rv.at[pl.ds(0, CD)],
                              code_hbm.at[pl.ds(0, CD)], sem).wait()

    def chunk_iter(c, carry):
        par = c & 1
        boff = par * CD
        cbase = c * CHUNK

        nxt = (wbase + cbase + CHUNK) * D
        nboff = (1 - par) * CD

        @pl.when((c + 1 < N_CHUNKS) & (par == 0))
        def _():
            pltpu.async_copy(eps_hbm.at[pl.ds(nxt, CD)],
                             eps_rv.at[pl.ds(nboff, CD)], sem_i1)

        @pl.when((c + 1 < N_CHUNKS) & (par == 1))
        def _():
            pltpu.async_copy(eps_hbm.at[pl.ds(nxt, CD)],
                             eps_rv.at[pl.ds(nboff, CD)], sem_i0)

        @pl.when(par == 0)
        def _():
            drain_in(sem_i0)

        @pl.when(par == 1)
        def _():
            drain_in(sem_i1)

        @pl.when((c >= 2) & (par == 0))
        def _():
            drain_out(sem_o0)

        @pl.when((c >= 2) & (par == 1))
        def _():
            drain_out(sem_o1)

        @plsc.parallel_loop(0, CHUNK // L, unroll=2)
        def group_body(g):
            gbase = cbase + g * L
            tbl_vec = idx_v[pl.ds(gbase, L)]
            kld_v[pl.ds(gbase, L)] = plsc.load_gather(g_v, [tbl_vec])
            for rr in range(L):
                toff = tbl_vec[rr] * D
                roff = boff + (g * L + rr) * D
                es = [eps_rv[pl.ds(roff + j * L, L)] for j in range(D // L)]
                sms = [sm_t[pl.ds(toff + j * L, L)] for j in range(D // L)]
                for j in range(D // L):
                    s, m = plsc.unpack(plsc.bitcast(sms[j], jnp.bfloat16),
                                       format=plsc.PackFormat.INTERLEAVED)
                    code_rv[pl.ds(roff + j * L, L)] = es[j] * s + m

        out = (wbase + cbase) * D

        @pl.when(par == 0)
        def _():
            pltpu.async_copy(code_rv.at[pl.ds(boff, CD)],
                             code_hbm.at[pl.ds(out, CD)], sem_o0)

        @pl.when(par == 1)
        def _():
            pltpu.async_copy(code_rv.at[pl.ds(boff, CD)],
                             code_hbm.at[pl.ds(out, CD)], sem_o1)

        return carry

    lax.fori_loop(0, N_CHUNKS, chunk_iter, 0)
    drain_out(sem_o0)
    drain_out(sem_o1)
    pltpu.sync_copy(kld_v, kld_hbm.at[pl.ds(wbase, B_PER_W)])


@jax.jit
def _run(indices, eps, mean_table, logvar_table):
    mesh = plsc.VectorSubcoreMesh(core_axis_name="c", subcore_axis_name="s")
    f = functools.partial(
        pl.kernel,
        out_type=(
            jax.ShapeDtypeStruct((B * D,), jnp.float32),
            jax.ShapeDtypeStruct((B,), jnp.float32),
        ),
        mesh=mesh,
        compiler_params=pltpu.CompilerParams(needs_layout_passes=False),
        scratch_types=[
            pltpu.VMEM_SHARED((VP * D,), jnp.int32),
            pltpu.VMEM_SHARED((VP,), jnp.float32),
            pltpu.VMEM((VP * D,), jnp.int32),
            pltpu.VMEM((VP,), jnp.float32),
            pltpu.VMEM((B_PER_W,), jnp.int32),
            pltpu.VMEM((B_PER_W,), jnp.float32),
            pltpu.VMEM((2 * CHUNK * D,), jnp.float32),
            pltpu.VMEM((2 * CHUNK * D,), jnp.float32),
            pltpu.VMEM((R_PER_S * D,), jnp.float32),
            pltpu.VMEM((R_PER_S * D,), jnp.float32),
            pltpu.VMEM((R_PER_S * D,), jnp.int32),
            pltpu.VMEM((L,), jnp.float32),
            pltpu.SemaphoreType.DMA,
            pltpu.SemaphoreType.DMA,
            pltpu.SemaphoreType.DMA,
            pltpu.SemaphoreType.DMA,
            pltpu.SemaphoreType.DMA,
        ],
    )(_sc_body)
    code_flat, kld = f(indices, eps.reshape(B * D),
                       mean_table.reshape(V * D),
                       logvar_table.reshape(V * D))
    return code_flat.reshape(B, D), kld


def kernel(indices, eps, mean_table, logvar_table):
    code, kld = _run(indices.astype(jnp.int32), eps, mean_table, logvar_table)
    return (code, kld)


# speedup vs baseline: 1.0161x; 1.0161x over previous
import functools

import jax
import jax.numpy as jnp
from jax import lax
from jax.experimental import pallas as pl
from jax.experimental.pallas import tpu as pltpu
from jax.experimental.pallas import tpu_sc as plsc

V = 100
VP = 128
D = 128
B = 16384
VARIATIONAL_WEIGHT = 0.001

NC = 2
NS = 16
L = 16
NW = NC * NS
B_PER_W = B // NW
CHUNK = 128
N_CHUNKS = B_PER_W // CHUNK
R_PER_S = 8
NS_FULL = V // R_PER_S


def _sc_body(idx_hbm, eps_hbm, mean_hbm, lv_hbm, code_hbm, kld_hbm,
             sm_sh, g_sh,
             sm_t, g_v, idx_v, kld_v,
             eps_rv, code_rv,
             a_st, b_st, p_st, g_st,
             sem_i0, sem_i1, sem_o0, sem_o1, sem_x):
    sid = lax.axis_index("s")
    wid = sid * NC + lax.axis_index("c")
    lane = lax.iota(jnp.int32, L)
    wbase = wid * B_PER_W
    CD = CHUNK * D

    cp_idx = pltpu.async_copy(idx_hbm.at[pl.ds(wbase, B_PER_W)], idx_v, sem_x)
    cp_eps0 = pltpu.async_copy(eps_hbm.at[pl.ds(wbase * D, CD)],
                               eps_rv.at[pl.ds(0, CD)], sem_i0)

    def prelude(trow, nrows):
        cp_a = pltpu.async_copy(lv_hbm.at[pl.ds(trow * D, nrows * D)],
                                a_st.at[pl.ds(0, nrows * D)], sem_x)
        cp_b = pltpu.async_copy(mean_hbm.at[pl.ds(trow * D, nrows * D)],
                                b_st.at[pl.ds(0, nrows * D)], sem_x)
        cp_a.wait()
        cp_b.wait()

        def prow(r2, g_acc):
            acc = jnp.zeros((L,), jnp.float32)
            for j in range(D // L):
                sl = pl.ds(r2 * D + j * L, L)
                lv = a_st[sl]
                m = b_st[sl]
                s = jnp.exp(0.5 * lv)
                p_st[pl.ds(r2 * D + j * L, L)] = plsc.bitcast(
                    plsc.pack(s, m, format=plsc.PackFormat.INTERLEAVED),
                    jnp.int32)
                acc = acc + (1.0 + lv - m * m - s * s)
            for _ in range(4):
                acc = acc + lax.rev(acc, (0,))
            return jnp.where(lane == r2, acc * (-0.5 * VARIATIONAL_WEIGHT),
                             g_acc)

        g_st[...] = lax.fori_loop(0, nrows, prow,
                                  jnp.zeros((L,), jnp.float32))
        pltpu.sync_copy(p_st.at[pl.ds(0, nrows * D)],
                        sm_sh.at[pl.ds(trow * D, nrows * D)])
        pltpu.sync_copy(g_st.at[pl.ds(0, R_PER_S)],
                        g_sh.at[pl.ds(trow, R_PER_S)])

    @pl.when(sid < NS_FULL)
    def _():
        prelude(sid * R_PER_S, R_PER_S)

    @pl.when(sid == NS_FULL)
    def _():
        prelude(NS_FULL * R_PER_S, V - NS_FULL * R_PER_S)

    plsc.subcore_barrier()
    cp_sm = pltpu.async_copy(sm_sh, sm_t, sem_x)
    cp_g = pltpu.async_copy(g_sh, g_v, sem_x)
    cp_idx.wait()
    cp_sm.wait()
    cp_g.wait()

    def drain_in(sem):
        pltpu.make_async_copy(eps_hbm.at[pl.ds(0, CD)],
                              eps_rv.at[pl.ds(0, CD)], sem).wait()

    def drain_out(sem):
        pltpu.make_async_copy(code_rv.at[pl.ds(0, CD)],
                              code_hbm.at[pl.ds(0, CD)], sem).wait()

    def chunk_iter(c, carry):
        par = c & 1
        boff = par * CD
        cbase = c * CHUNK

        nxt = (wbase + cbase + CHUNK) * D
        nboff = (1 - par) * CD

        @pl.when((c + 1 < N_CHUNKS) & (par == 0))
        def _():
            pltpu.async_copy(eps_hbm.at[pl.ds(nxt, CD)],
                             eps_rv.at[pl.ds(nboff, CD)], sem_i1)

        @pl.when((c + 1 < N_CHUNKS) & (par == 1))
        def _():
            pltpu.async_copy(eps_hbm.at[pl.ds(nxt, CD)],
                             eps_rv.at[pl.ds(nboff, CD)], sem_i0)

        @pl.when(par == 0)
        def _():
            drain_in(sem_i0)

        @pl.when(par == 1)
        def _():
            drain_in(sem_i1)

        @pl.when((c >= 2) & (par == 0))
        def _():
            drain_out(sem_o0)

        @pl.when((c >= 2) & (par == 1))
        def _():
            drain_out(sem_o1)

        @plsc.parallel_loop(0, CHUNK // L)
        def group_body(g):
            gbase = cbase + g * L
            tbl_vec = idx_v[pl.ds(gbase, L)]
            kld_v[pl.ds(gbase, L)] = plsc.load_gather(g_v, [tbl_vec])
            for rr in range(L):
                toff = tbl_vec[rr] * D
                roff = boff + (g * L + rr) * D
                es = [eps_rv[pl.ds(roff + j * L, L)] for j in range(D // L)]
                sms = [sm_t[pl.ds(toff + j * L, L)] for j in range(D // L)]
                for j in range(D // L):
                    s, m = plsc.unpack(plsc.bitcast(sms[j], jnp.bfloat16),
                                       format=plsc.PackFormat.INTERLEAVED)
                    code_rv[pl.ds(roff + j * L, L)] = es[j] * s + m

        out = (wbase + cbase) * D

        @pl.when(par == 0)
        def _():
            pltpu.async_copy(code_rv.at[pl.ds(boff, CD)],
                             code_hbm.at[pl.ds(out, CD)], sem_o0)

        @pl.when(par == 1)
        def _():
            pltpu.async_copy(code_rv.at[pl.ds(boff, CD)],
                             code_hbm.at[pl.ds(out, CD)], sem_o1)

        return carry

    lax.fori_loop(0, N_CHUNKS, chunk_iter, 0)
    drain_out(sem_o0)
    drain_out(sem_o1)
    pltpu.sync_copy(kld_v, kld_hbm.at[pl.ds(wbase, B_PER_W)])


@jax.jit
def _run(indices, eps, mean_table, logvar_table):
    mesh = plsc.VectorSubcoreMesh(core_axis_name="c", subcore_axis_name="s")
    f = functools.partial(
        pl.kernel,
        out_type=(
            jax.ShapeDtypeStruct((B * D,), jnp.float32),
            jax.ShapeDtypeStruct((B,), jnp.float32),
        ),
        mesh=mesh,
        compiler_params=pltpu.CompilerParams(needs_layout_passes=False),
        scratch_types=[
            pltpu.VMEM_SHARED((VP * D,), jnp.int32),
            pltpu.VMEM_SHARED((VP,), jnp.float32),
            pltpu.VMEM((VP * D,), jnp.int32),
            pltpu.VMEM((VP,), jnp.float32),
            pltpu.VMEM((B_PER_W,), jnp.int32),
            pltpu.VMEM((B_PER_W,), jnp.float32),
            pltpu.VMEM((2 * CHUNK * D,), jnp.float32),
            pltpu.VMEM((2 * CHUNK * D,), jnp.float32),
            pltpu.VMEM((R_PER_S * D,), jnp.float32),
            pltpu.VMEM((R_PER_S * D,), jnp.float32),
            pltpu.VMEM((R_PER_S * D,), jnp.int32),
            pltpu.VMEM((L,), jnp.float32),
            pltpu.SemaphoreType.DMA,
            pltpu.SemaphoreType.DMA,
            pltpu.SemaphoreType.DMA,
            pltpu.SemaphoreType.DMA,
            pltpu.SemaphoreType.DMA,
        ],
    )(_sc_body)
    code_flat, kld = f(indices, eps.reshape(B * D),
                       mean_table.reshape(V * D),
                       logvar_table.reshape(V * D))
    return code_flat.reshape(B, D), kld


def kernel(indices, eps, mean_table, logvar_table):
    code, kld = _run(indices.astype(jnp.int32), eps, mean_table, logvar_table)
    return (code, kld)
